# baseline (speedup 1.0000x reference)
import functools

import jax
import jax.numpy as jnp
import numpy as np
from jax import lax
from jax.experimental import pallas as pl
from jax.experimental.pallas import tpu as pltpu
from jax.experimental.pallas import tpu_sc as plsc

_PNS = (1, 2, 4, 8, 16, 32, 64, 128, 256, 512)
_VOCAB = 8192
_C = 256
_B = 16
_H = 512
_BETA = 0.25
_SHARE = 4
_NSC = 10


def _cubic_w_np(t, a=-0.75):
    at = np.abs(t)
    w1 = (a + 2.0) * at ** 3 - (a + 3.0) * at ** 2 + 1.0
    w2 = a * at ** 3 - 5.0 * a * at ** 2 + 8.0 * a * at - 4.0 * a
    return np.where(at <= 1.0, w1, np.where(at < 2.0, w2, np.zeros_like(at)))


@functools.lru_cache(maxsize=None)
def _upsample_weights(pn: int, out_h: int) -> np.ndarray:
    scale = pn / out_h
    i = np.arange(out_h, dtype=np.float32)
    src = (i + 0.5) * scale - 0.5
    i0 = np.floor(src).astype(np.int32)
    w = np.zeros((out_h, 4), dtype=np.float32)
    for t in range(4):
        tap = i0 - 1 + t
        w[:, t] = _cubic_w_np((src - tap).astype(np.float32))
    return w


@functools.lru_cache(maxsize=None)
def _phi_share(si: int) -> int:
    ticks = np.linspace(1.0 / 3.0 / _SHARE, 1.0 - 1.0 / 3.0 / _SHARE, _SHARE)
    return int(np.argmin(np.abs(ticks - si / (_NSC - 1))))


def _prologue_kernel(cb_ref, f_ref, cbn_ref, ds_ref, fr_ref):
    cb = cb_ref[...]
    nrm = jnp.sqrt(jnp.sum(cb * cb, axis=1))
    cbn_ref[...] = cb / jnp.maximum(nrm, 1e-12)[:, None]
    fb = f_ref[0]
    ds_ref[0, 0, :] = jnp.mean(fb, axis=1)
    fr_ref[0] = jnp.transpose(fb)


def _prologue(codebook, f_nat):
    return pl.pallas_call(
        _prologue_kernel,
        grid=(_B,),
        in_specs=[
            pl.BlockSpec((_VOCAB // _B, _C), lambda i: (i, 0)),
            pl.BlockSpec((1, _C, _H), lambda i: (i, 0, 0)),
        ],
        out_specs=[
            pl.BlockSpec((_VOCAB // _B, _C), lambda i: (i, 0)),
            pl.BlockSpec((1, 1, _C), lambda i: (i, 0, 0)),
            pl.BlockSpec((1, _H, _C), lambda i: (i, 0, 0)),
        ],
        out_shape=[
            jax.ShapeDtypeStruct((_VOCAB, _C), jnp.float32),
            jax.ShapeDtypeStruct((_B, 1, _C), jnp.float32),
            jax.ShapeDtypeStruct((_B, _H, _C), jnp.float32),
        ],
    )(codebook, f_nat)


def _normalize_rows(q):
    return q / jnp.maximum(jnp.sqrt(jnp.sum(q * q, axis=1)), 1e-12)[:, None]


def _score(qn, cbn):
    return lax.dot_general(qn.astype(jnp.bfloat16), cbn.astype(jnp.bfloat16),
                           (((1,), (1,)), ((), ())),
                           preferred_element_type=jnp.float32)


def _argmax0_kernel(q_ref, cbn_ref, idx_ref):
    qn = _normalize_rows(q_ref[...])
    s = _score(qn, cbn_ref[...])
    idx_ref[0, 0, :] = jnp.argmax(s, axis=1).astype(jnp.int32)


def _argmax0(ds0, cbn):
    return pl.pallas_call(
        _argmax0_kernel,
        out_specs=pl.BlockSpec((1, 1, _B), lambda: (0, 0, 0)),
        out_shape=jax.ShapeDtypeStruct((1, 1, _B), jnp.int32),
    )(ds0, cbn)


def _sc_gather(codebook, idx):
    info = plsc.get_sparse_core_info()
    nw = info.num_cores * info.num_subcores
    n = idx.shape[0]
    nw_used = min(nw, n // 8)
    b_per_w = n // nw_used
    mesh = plsc.VectorSubcoreMesh(core_axis_name="c", subcore_axis_name="s")

    @functools.partial(
        pl.kernel, mesh=mesh,
        out_type=jax.ShapeDtypeStruct((n, _C), jnp.float32),
        scratch_types=[
            pltpu.VMEM((b_per_w,), jnp.int32),
            pltpu.VMEM((b_per_w, _C), jnp.float32),
            pltpu.SemaphoreType.DMA,
        ],
    )
    def k(table_hbm, idx_hbm, out_hbm, idx_v, rows_v, sem):
        wid = lax.axis_index("s") * info.num_cores + lax.axis_index("c")

        @pl.when(wid < nw_used)
        def _():
            base = wid * b_per_w
            pltpu.sync_copy(idx_hbm.at[pl.ds(base, b_per_w)], idx_v)
            pltpu.async_copy(table_hbm.at[idx_v], rows_v, sem).wait()
            pltpu.sync_copy(rows_v, out_hbm.at[pl.ds(base, b_per_w)])

    return k(codebook, idx)


def _shift_clamp(x, off):
    n = x.shape[0]
    if off == 0:
        return x
    if off > 0:
        o = min(off, n)
        edge = jnp.broadcast_to(x[0:1, :], (o, x.shape[1]))
        if o == n:
            return edge
        return jnp.concatenate([edge, x[: n - o, :]], axis=0)
    o = min(-off, n)
    edge = jnp.broadcast_to(x[n - 1 : n, :], (o, x.shape[1]))
    if o == n:
        return edge
    return jnp.concatenate([x[o:, :], edge], axis=0)


def _upsample_vpu(pn, g, uw_ref):
    r = _H // pn
    g_exp = jnp.broadcast_to(g[:, None, :], (pn, r, _C)).reshape(_H, _C)
    gp = None
    for t in range(4):
        off = r // 2 - (t - 1) * r
        term = uw_ref[:, t][:, None] * _shift_clamp(g_exp, off)
        gp = term if gp is None else gp + term
    return gp


def _phi_update(si, g, uw_ref, w_ref, b_ref, rest_blk):
    if si != _NSC - 1:
        gp = _upsample_vpu(_PNS[si], g, uw_ref)
    else:
        gp = g
    gpb = gp.astype(jnp.bfloat16)
    zrow = jnp.zeros((1, _C), jnp.bfloat16)
    sd = jnp.concatenate([zrow, gpb[:-1, :]], axis=0)
    su = jnp.concatenate([gpb[1:, :], zrow], axis=0)
    mm = lambda x, w: lax.dot_general(
        x, w, (((1,), (0,)), ((), ())), preferred_element_type=jnp.float32)
    w = w_ref[...].astype(jnp.bfloat16)
    y2 = mm(sd, w[0]) + mm(gpb, w[1]) + mm(su, w[2])
    h = 0.5 * gp + 0.5 * (y2 + b_ref[0, :][None, :])
    return rest_blk - h


def _merged_kernel(si, pn, pn_next, bb, nrb, g_ref, uw_ref, w_ref, b_ref,
                   rest_ref, cbn_ref, rest_out, idx_out, ss_out):
    i = pl.program_id(0) if nrb > 1 else 0
    r_next = _H // pn_next
    ss = None
    ds_list = []
    for b in range(bb):
        rnew = _phi_update(si, g_ref[b], uw_ref, w_ref, b_ref, rest_ref[b])
        rest_out[b] = rnew
        if r_next > 1:
            ds_list.append(jnp.mean(rnew.reshape(pn_next, r_next, _C), axis=1))
        else:
            ds_list.append(rnew)
        ssq = jnp.sum(rnew * rnew)
        ss = ssq if ss is None else ss + ssq

    if nrb == 1:
        ss_out[0, 0] = ss
    else:
        @pl.when(i == 0)
        def _():
            ss_out[0, 0] = ss

        @pl.when(i > 0)
        def _():
            ss_out[0, 0] += ss

    qn = _normalize_rows(jnp.concatenate(ds_list, axis=0)
                         if bb > 1 else ds_list[0])
    cbs = 2048
    m = None
    a = None
    for k in range(_VOCAB // cbs):
        s = _score(qn, cbn_ref[k * cbs:(k + 1) * cbs, :])
        lmax = jnp.max(s, axis=1)
        larg = jnp.argmax(s, axis=1).astype(jnp.int32) + k * cbs
        if m is None:
            m, a = lmax, larg
        else:
            better = lmax > m
            m = jnp.where(better, lmax, m)
            a = jnp.where(better, larg, a)
    idx_out[0, 0, :] = a


def _merged(si, g, rest, cbn, uw, w3, bias):
    pn = _PNS[si]
    pn_next = _PNS[si + 1]
    bb = min(_B, max(1, 1024 // pn_next))
    nrb = _B // bb
    qrows = bb * pn_next

    body = functools.partial(_merged_kernel, si, pn, pn_next, bb, nrb)
    return pl.pallas_call(
        body,
        grid=(nrb,),
        in_specs=[
            pl.BlockSpec((bb, pn, _C), lambda i: (i, 0, 0)),
            pl.BlockSpec((_H, 4), lambda i: (0, 0)),
            pl.BlockSpec((3, _C, _C), lambda i: (0, 0, 0)),
            pl.BlockSpec((1, _C), lambda i: (0, 0)),
            pl.BlockSpec((bb, _H, _C), lambda i: (i, 0, 0)),
            pl.BlockSpec((_VOCAB, _C), lambda i: (0, 0)),
        ],
        out_specs=[
            pl.BlockSpec((bb, _H, _C), lambda i: (i, 0, 0)),
            pl.BlockSpec((1, 1, qrows), lambda i: (i, 0, 0)),
            pl.BlockSpec((1, 1), lambda i: (0, 0), memory_space=pltpu.SMEM),
        ],
        out_shape=[
            jax.ShapeDtypeStruct((_B, _H, _C), jnp.float32),
            jax.ShapeDtypeStruct((nrb, 1, qrows), jnp.int32),
            jax.ShapeDtypeStruct((1, 1), jnp.float32),
        ],
    )(g.reshape(_B, pn, _C), uw, w3, bias, rest, cbn)


def _last_kernel(g_ref, w_ref, b_ref, rest_ref, f_ref, ss_out, fhat_out):
    ss = None
    for b in range(_B):
        rnew = _phi_update(_NSC - 1, g_ref[b], None, w_ref, b_ref, rest_ref[b])
        fhat_out[b] = f_ref[b] - jnp.transpose(rnew)
        ssq = jnp.sum(rnew * rnew)
        ss = ssq if ss is None else ss + ssq
    ss_out[0, 0] = ss


def _update_last(g, rest, f_nat, w3, bias):
    pn = _PNS[-1]
    return pl.pallas_call(
        _last_kernel,
        in_specs=[
            pl.BlockSpec((_B, pn, _C), lambda: (0, 0, 0)),
            pl.BlockSpec((3, _C, _C), lambda: (0, 0, 0)),
            pl.BlockSpec((1, _C), lambda: (0, 0)),
            pl.BlockSpec((_B, _H, _C), lambda: (0, 0, 0)),
            pl.BlockSpec((_B, _C, _H), lambda: (0, 0, 0)),
        ],
        out_specs=[
            pl.BlockSpec((1, 1), lambda: (0, 0), memory_space=pltpu.SMEM),
            pl.BlockSpec((_B, _C, _H), lambda: (0, 0, 0)),
        ],
        out_shape=[
            jax.ShapeDtypeStruct((1, 1), jnp.float32),
            jax.ShapeDtypeStruct((_B, _C, _H), jnp.float32),
        ],
    )(g.reshape(_B, pn, _C), w3, bias, rest, f_nat)


def kernel(f_BChw, codebook, phi_w, phi_b):
    f_nat = f_BChw.reshape(_B, _C, _H)

    cbn, ds0, f_r = _prologue(codebook, f_nat)

    w3s, biases = [], []
    for si in range(_NSC):
        k = _phi_share(si)
        w3s.append(jnp.transpose(phi_w[k, :, :, :, 1], (2, 1, 0)))
        biases.append(phi_b[k].reshape(1, _C))

    rest = f_r
    idx = _argmax0(ds0.reshape(_B, _C), cbn).reshape(_B)
    ss_list = []
    for si in range(_NSC - 1):
        g = _sc_gather(codebook, idx)
        uw = jnp.asarray(_upsample_weights(_PNS[si], _H))
        rest, idx3, ss = _merged(si, g, rest, cbn, uw, w3s[si], biases[si])
        idx = idx3.reshape(_B * _PNS[si + 1])
        ss_list.append(ss[0, 0])

    g = _sc_gather(codebook, idx)
    ss9, fhat = _update_last(g, rest, f_nat, w3s[-1], biases[-1])
    ss_list.append(ss9[0, 0])

    numel = _B * _H * _C
    loss = (1.0 + _BETA) / _NSC * jnp.sum(jnp.stack(ss_list)) / numel
    f_hat_out = fhat.reshape(_B, _C, _H, 1)
    return (f_hat_out, loss)
